# baseline (speedup 1.0000x reference)
import functools
import math

import jax
import jax.numpy as jnp
from jax import lax
from jax.experimental import pallas as pl
from jax.experimental.pallas import tpu as pltpu
from jax.experimental.pallas import tpu_sc as plsc

LANES = 16


@functools.lru_cache(maxsize=None)
def _make_sc_gather(n_rows: int, d: int, k: int):
    info = plsc.get_sparse_core_info()
    nc, ns = info.num_cores, info.num_subcores
    nw = nc * ns
    assert n_rows % (nw * k) == 0
    rows_per_w = n_rows // nw
    n_chunks = rows_per_w // k
    assert n_chunks % 2 == 0 and n_chunks >= 6
    scale = math.sqrt(float(d))
    mesh = plsc.VectorSubcoreMesh(core_axis_name="c", subcore_axis_name="s")

    @functools.partial(
        pl.kernel,
        mesh=mesh,
        out_type=jax.ShapeDtypeStruct((n_rows, d), jnp.float32),
        scratch_types=[
            pltpu.VMEM((n_chunks, k), jnp.int32),
            pltpu.VMEM((2, k, d), jnp.float32),
            pltpu.VMEM((2, k, d), jnp.float32),
            pltpu.SemaphoreType.DMA,
            pltpu.SemaphoreType.DMA,
            pltpu.SemaphoreType.DMA,
            pltpu.SemaphoreType.DMA,
        ],
    )
    def gather_scale(idx_hbm, table_hbm, out_hbm, idx_v, gbuf, sbuf,
                     sem_g0, sem_g1, sem_s0, sem_s1):
        wid = lax.axis_index("s") * nc + lax.axis_index("c")
        base = wid * rows_per_w
        sems_g = (sem_g0, sem_g1)
        sems_s = (sem_s0, sem_s1)

        pltpu.sync_copy(idx_hbm.at[wid], idx_v)

        def issue_gather(cc, b):
            pltpu.async_copy(table_hbm.at[idx_v.at[cc]], gbuf.at[b], sems_g[b])

        def wait_gather(b):
            pltpu.make_async_copy(table_hbm.at[idx_v.at[0]], gbuf.at[b],
                                  sems_g[b]).wait()

        def issue_scatter(cc, b):
            pltpu.async_copy(sbuf.at[b], out_hbm.at[pl.ds(base + cc * k, k)],
                             sems_s[b])

        def wait_scatter(b):
            pltpu.make_async_copy(sbuf.at[b], out_hbm.at[pl.ds(base, k)],
                                  sems_s[b]).wait()

        def scale_chunk(b):
            def slot(j, carry):
                for r in range(k):
                    sbuf[b, r, pl.ds(j * LANES, LANES)] = (
                        gbuf[b, r, pl.ds(j * LANES, LANES)] * scale
                    )
                return carry
            lax.fori_loop(0, d // LANES, slot, 0, unroll=4)

        issue_gather(0, 0)
        issue_gather(1, 1)

        for b in range(2):
            wait_gather(b)
            scale_chunk(b)
            issue_gather(2 + b, b)
            issue_scatter(b, b)

        def group(g, carry):
            cc0 = g * 2
            for b in range(2):
                cc = cc0 + b
                wait_gather(b)
                wait_scatter(b)
                scale_chunk(b)
                issue_gather(cc + 2, b)
                issue_scatter(cc, b)
            return carry

        lax.fori_loop(1, n_chunks // 2 - 1, group, 0, unroll=False)

        for b in range(2):
            cc = n_chunks - 2 + b
            wait_gather(b)
            wait_scatter(b)
            scale_chunk(b)
            issue_scatter(cc, b)

        for b in range(2):
            wait_scatter(b)

    return gather_scale


def kernel(sequence, table):
    b, s = sequence.shape
    vocab, d = table.shape
    n_rows = b * s
    k = 8
    info = plsc.get_sparse_core_info()
    nw = info.num_cores * info.num_subcores
    idx = sequence.reshape(nw, (n_rows // nw) // k, k).astype(jnp.int32)
    fn = _make_sc_gather(n_rows, d, k)
    out = fn(idx, table)
    return out.reshape(b, s, d)


# speedup vs baseline: 1.9672x; 1.9672x over previous
import functools
import math

import jax
import jax.numpy as jnp
from jax import lax
from jax.experimental import pallas as pl
from jax.experimental.pallas import tpu as pltpu
from jax.experimental.pallas import tpu_sc as plsc

LANES = 16


@functools.lru_cache(maxsize=None)
def _make_sc_gather(n_rows: int, d: int, k: int):
    info = plsc.get_sparse_core_info()
    nc, ns = info.num_cores, info.num_subcores
    nw = nc * ns
    assert n_rows % (nw * k) == 0
    rows_per_w = n_rows // nw
    n_chunks = rows_per_w // k
    assert n_chunks % 2 == 0 and n_chunks >= 6
    scale = math.sqrt(float(d))
    mesh = plsc.VectorSubcoreMesh(core_axis_name="c", subcore_axis_name="s")

    @functools.partial(
        pl.kernel,
        mesh=mesh,
        out_type=jax.ShapeDtypeStruct((n_rows, d), jnp.float32),
        scratch_types=[
            pltpu.VMEM((n_chunks, k), jnp.int32),
            pltpu.VMEM((2, k, d), jnp.float32),
            pltpu.VMEM((2, k, d), jnp.float32),
            pltpu.SemaphoreType.DMA,
            pltpu.SemaphoreType.DMA,
            pltpu.SemaphoreType.DMA,
            pltpu.SemaphoreType.DMA,
        ],
    )
    def gather_scale(idx_hbm, table_hbm, out_hbm, idx_v, gbuf, sbuf,
                     sem_g0, sem_g1, sem_s0, sem_s1):
        wid = lax.axis_index("s") * nc + lax.axis_index("c")
        base = wid * rows_per_w
        sems_g = (sem_g0, sem_g1)
        sems_s = (sem_s0, sem_s1)

        pltpu.sync_copy(idx_hbm.at[wid], idx_v)

        def issue_gather(cc, b):
            pltpu.async_copy(table_hbm.at[idx_v.at[cc]], gbuf.at[b], sems_g[b])

        def wait_gather(b):
            pltpu.make_async_copy(table_hbm.at[idx_v.at[0]], gbuf.at[b],
                                  sems_g[b]).wait()

        def issue_scatter(cc, b):
            pltpu.async_copy(sbuf.at[b], out_hbm.at[pl.ds(base + cc * k, k)],
                             sems_s[b])

        def wait_scatter(b):
            pltpu.make_async_copy(sbuf.at[b], out_hbm.at[pl.ds(base, k)],
                                  sems_s[b]).wait()

        def scale_chunk(b):
            @plsc.parallel_loop(0, d // LANES, unroll=4)
            def _(j):
                for r in range(k):
                    sbuf[b, r, pl.ds(j * LANES, LANES)] = (
                        gbuf[b, r, pl.ds(j * LANES, LANES)] * scale
                    )

        issue_gather(0, 0)
        issue_gather(1, 1)

        for b in range(2):
            wait_gather(b)
            scale_chunk(b)
            issue_gather(2 + b, b)
            issue_scatter(b, b)

        def group(g, carry):
            cc0 = g * 2
            for b in range(2):
                cc = cc0 + b
                wait_gather(b)
                wait_scatter(b)
                scale_chunk(b)
                issue_gather(cc + 2, b)
                issue_scatter(cc, b)
            return carry

        lax.fori_loop(1, n_chunks // 2 - 1, group, 0, unroll=False)

        for b in range(2):
            cc = n_chunks - 2 + b
            wait_gather(b)
            wait_scatter(b)
            scale_chunk(b)
            issue_scatter(cc, b)

        for b in range(2):
            wait_scatter(b)

    return gather_scale


def kernel(sequence, table):
    b, s = sequence.shape
    vocab, d = table.shape
    n_rows = b * s
    k = 8
    info = plsc.get_sparse_core_info()
    nw = info.num_cores * info.num_subcores
    idx = sequence.reshape(nw, (n_rows // nw) // k, k).astype(jnp.int32)
    fn = _make_sc_gather(n_rows, d, k)
    out = fn(idx, table)
    return out.reshape(b, s, d)
